# Initial kernel scaffold; baseline (speedup 1.0000x reference)
import jax
import jax.numpy as jnp
from jax.experimental import pallas as pl


def kernel(X, edge_index, D, Wp1, bp1, Ws1, bs1, Wpe1, bpe1, Wse1, bse1, Wp2, bp2, Ws2, bs2, Wpe2, bpe2, Wse2, bse2, Wc, bc):
    raise NotImplementedError("write your pallas kernel here")


# speedup vs baseline: 2.4644x; 2.4644x over previous
import functools

import jax
import jax.numpy as jnp
from jax import lax
from jax.experimental import pallas as pl
from jax.experimental.pallas import tpu as pltpu
from jax.experimental.pallas import tpu_sc as plsc

NN = 10000
EE = 320000
DF = 128
HH = 32

NC = 2
NS = 16
NW = NC * NS
EPW = EE // NW
CHK = 80
NCHUNK = EPW // CHK
NPAD = 10240
RPT = NPAD // NS
ZR = 128

_MESH = dict(core_axis_name="c", subcore_axis_name="s")


def _sc1_body(x_hbm, src_hbm, dst_hbm, w0_hbm, aggp_hbm,
              idx_s, idx_d, xs, xd, w0, zbuf, sem_s, sem_d, acc):
    core = lax.axis_index("c")
    sid = lax.axis_index("s")
    wid = core * NS + sid
    ebase = wid * EPW
    row0 = sid * RPT

    def zrow(i, _):
        for k in range(0, DF, 16):
            zbuf[i, pl.ds(k, 16)] = jnp.zeros((16,), jnp.float32)
        return 0
    lax.fori_loop(0, ZR, zrow, 0)
    for j in range(RPT // ZR):
        pltpu.sync_copy(zbuf, acc.at[pl.ds(row0 + j * ZR, ZR)])
    plsc.subcore_barrier()

    def chunk(c, _):
        base = ebase + c * CHK
        pltpu.sync_copy(src_hbm.at[pl.ds(base, CHK)], idx_s)
        pltpu.sync_copy(dst_hbm.at[pl.ds(base, CHK)], idx_d)
        cp_s = pltpu.async_copy(x_hbm.at[idx_s], xs, sem_s)
        cp_d = pltpu.async_copy(x_hbm.at[idx_d], xd, sem_d)
        cp_s.wait()
        cp_d.wait()

        def row(i, _):
            for k in range(0, DF, 16):
                sl = (i, pl.ds(k, 16))
                w0[sl] = jnp.abs(xs[sl] - xd[sl])
            return 0
        lax.fori_loop(0, CHK, row, 0)

        pltpu.sync_copy(w0, w0_hbm.at[pl.ds(base, CHK)])
        pltpu.sync_copy(w0, acc.at[idx_s], add=True)
        return 0
    lax.fori_loop(0, NCHUNK, chunk, 0)

    plsc.subcore_barrier()
    pltpu.sync_copy(acc.at[pl.ds(row0, RPT)],
                    aggp_hbm.at[core, pl.ds(row0, RPT)])


_sc1 = pl.kernel(
    _sc1_body,
    out_type=[jax.ShapeDtypeStruct((EE, DF), jnp.float32),
              jax.ShapeDtypeStruct((2, NPAD, DF), jnp.float32)],
    mesh=plsc.VectorSubcoreMesh(**_MESH),
    scratch_types=[
        pltpu.VMEM((CHK,), jnp.int32),
        pltpu.VMEM((CHK,), jnp.int32),
        pltpu.VMEM((CHK, DF), jnp.float32),
        pltpu.VMEM((CHK, DF), jnp.float32),
        pltpu.VMEM((CHK, DF), jnp.float32),
        pltpu.VMEM((ZR, DF), jnp.float32),
        pltpu.SemaphoreType.DMA,
        pltpu.SemaphoreType.DMA,
        pltpu.VMEM_SHARED((NPAD, DF), jnp.float32),
    ],
)


def _sc_edge_body(with_acc, g1_hbm, g2_hbm, u_hbm, src_hbm, dst_hbm,
                  w_hbm, *rest):
    if with_acc:
        (aggp_hbm, idx_s, idx_d, b1, b2, bu, bw, zbuf,
         sem1, sem2, sem3, acc) = rest
    else:
        (idx_s, idx_d, b1, b2, bu, bw, sem1, sem2, sem3) = rest
    core = lax.axis_index("c")
    sid = lax.axis_index("s")
    wid = core * NS + sid
    ebase = wid * EPW
    row0 = sid * RPT

    if with_acc:
        def zrow(i, _):
            for k in range(0, HH, 16):
                zbuf[i, pl.ds(k, 16)] = jnp.zeros((16,), jnp.float32)
            return 0
        lax.fori_loop(0, ZR, zrow, 0)
        for j in range(RPT // ZR):
            pltpu.sync_copy(zbuf, acc.at[pl.ds(row0 + j * ZR, ZR)])
        plsc.subcore_barrier()

    def chunk(c, _):
        base = ebase + c * CHK
        pltpu.sync_copy(src_hbm.at[pl.ds(base, CHK)], idx_s)
        pltpu.sync_copy(dst_hbm.at[pl.ds(base, CHK)], idx_d)
        cp1 = pltpu.async_copy(g1_hbm.at[idx_s], b1, sem1)
        cp2 = pltpu.async_copy(g2_hbm.at[idx_d], b2, sem2)
        cp3 = pltpu.async_copy(u_hbm.at[pl.ds(base, CHK)], bu, sem3)
        cp1.wait()
        cp2.wait()
        cp3.wait()

        def row(i, _):
            for k in range(0, HH, 16):
                sl = (i, pl.ds(k, 16))
                bw[sl] = jnp.maximum(b1[sl] + b2[sl] + bu[sl], 0.0)
            return 0
        lax.fori_loop(0, CHK, row, 0)

        pltpu.sync_copy(bw, w_hbm.at[pl.ds(base, CHK)])
        if with_acc:
            pltpu.sync_copy(bw, acc.at[idx_s], add=True)
        return 0
    lax.fori_loop(0, NCHUNK, chunk, 0)

    if with_acc:
        plsc.subcore_barrier()
        pltpu.sync_copy(acc.at[pl.ds(row0, RPT)],
                        aggp_hbm.at[core, pl.ds(row0, RPT)])


_sc2 = pl.kernel(
    functools.partial(_sc_edge_body, True),
    out_type=[jax.ShapeDtypeStruct((EE, HH), jnp.float32),
              jax.ShapeDtypeStruct((2, NPAD, HH), jnp.float32)],
    compiler_params=pltpu.CompilerParams(use_tc_tiling_on_sc=False),
    mesh=plsc.VectorSubcoreMesh(**_MESH),
    scratch_types=[
        pltpu.VMEM((CHK,), jnp.int32),
        pltpu.VMEM((CHK,), jnp.int32),
        pltpu.VMEM((CHK, HH), jnp.float32),
        pltpu.VMEM((CHK, HH), jnp.float32),
        pltpu.VMEM((CHK, HH), jnp.float32),
        pltpu.VMEM((CHK, HH), jnp.float32),
        pltpu.VMEM((ZR, HH), jnp.float32),
        pltpu.SemaphoreType.DMA,
        pltpu.SemaphoreType.DMA,
        pltpu.SemaphoreType.DMA,
        pltpu.VMEM_SHARED((NPAD, HH), jnp.float32),
    ],
)

_sc3 = pl.kernel(
    functools.partial(_sc_edge_body, False),
    out_type=[jax.ShapeDtypeStruct((EE, HH), jnp.float32)],
    compiler_params=pltpu.CompilerParams(use_tc_tiling_on_sc=False),
    mesh=plsc.VectorSubcoreMesh(**_MESH),
    scratch_types=[
        pltpu.VMEM((CHK,), jnp.int32),
        pltpu.VMEM((CHK,), jnp.int32),
        pltpu.VMEM((CHK, HH), jnp.float32),
        pltpu.VMEM((CHK, HH), jnp.float32),
        pltpu.VMEM((CHK, HH), jnp.float32),
        pltpu.VMEM((CHK, HH), jnp.float32),
        pltpu.SemaphoreType.DMA,
        pltpu.SemaphoreType.DMA,
        pltpu.SemaphoreType.DMA,
    ],
)


_NBLK = 1000
_NGRID = NN // _NBLK
_EBLK = 2560
_EGRID = EE // _EBLK


def _node1_body(a0, a1, d, wp, wa, wb, c0, xn_o, g1_o, g2_o):
    agg = a0[0] + a1[0]
    xn = jnp.maximum(
        jnp.dot(agg / d[...], wp[...], preferred_element_type=jnp.float32)
        + c0[...], 0.0)
    xn_o[...] = xn
    g1_o[...] = jnp.dot(xn, wa[...], preferred_element_type=jnp.float32)
    g2_o[...] = jnp.dot(xn, wb[...], preferred_element_type=jnp.float32)


def _tc_node1(aggp, d2, wp1t, wa1t, wb1t, c0):
    return pl.pallas_call(
        _node1_body,
        grid=(_NGRID,),
        in_specs=[
            pl.BlockSpec((1, _NBLK, DF), lambda i: (0, i, 0)),
            pl.BlockSpec((1, _NBLK, DF), lambda i: (1, i, 0)),
            pl.BlockSpec((_NBLK, 1), lambda i: (i, 0)),
            pl.BlockSpec((DF, HH), lambda i: (0, 0)),
            pl.BlockSpec((HH, HH), lambda i: (0, 0)),
            pl.BlockSpec((HH, HH), lambda i: (0, 0)),
            pl.BlockSpec((1, HH), lambda i: (0, 0)),
        ],
        out_specs=[
            pl.BlockSpec((_NBLK, HH), lambda i: (i, 0)),
            pl.BlockSpec((_NBLK, HH), lambda i: (i, 0)),
            pl.BlockSpec((_NBLK, HH), lambda i: (i, 0)),
        ],
        out_shape=[jax.ShapeDtypeStruct((NN, HH), jnp.float32)] * 3,
    )(aggp, aggp, d2, wp1t, wa1t, wb1t, c0)


def _node2_body(a0, a1, d, xn, wp, ws, c0, wa, wb, g3_o, g4_o):
    agg = a0[0] + a1[0]
    xn2 = jnp.maximum(
        jnp.dot(agg / d[...], wp[...], preferred_element_type=jnp.float32)
        + jnp.dot(xn[...], ws[...], preferred_element_type=jnp.float32)
        + c0[...], 0.0)
    g3_o[...] = jnp.dot(xn2, wa[...], preferred_element_type=jnp.float32)
    g4_o[...] = jnp.dot(xn2, wb[...], preferred_element_type=jnp.float32)


def _tc_node2(agg2p, d2, xn, wp2t, ws2t, c2, wa2t, wb2t):
    hblk = pl.BlockSpec((HH, HH), lambda i: (0, 0))
    return pl.pallas_call(
        _node2_body,
        grid=(_NGRID,),
        in_specs=[
            pl.BlockSpec((1, _NBLK, HH), lambda i: (0, i, 0)),
            pl.BlockSpec((1, _NBLK, HH), lambda i: (1, i, 0)),
            pl.BlockSpec((_NBLK, 1), lambda i: (i, 0)),
            pl.BlockSpec((_NBLK, HH), lambda i: (i, 0)),
            hblk, hblk,
            pl.BlockSpec((1, HH), lambda i: (0, 0)),
            hblk, hblk,
        ],
        out_specs=[
            pl.BlockSpec((_NBLK, HH), lambda i: (i, 0)),
            pl.BlockSpec((_NBLK, HH), lambda i: (i, 0)),
        ],
        out_shape=[jax.ShapeDtypeStruct((NN, HH), jnp.float32)] * 2,
    )(agg2p, agg2p, d2, xn, wp2t, ws2t, c2, wa2t, wb2t)


def _edge_mm_body(w, m, b, o):
    o[...] = jnp.dot(w[...], m[...], preferred_element_type=jnp.float32) + b[...]


def _tc_edge_mm(w, m, b, kdim):
    return pl.pallas_call(
        _edge_mm_body,
        grid=(_EGRID,),
        in_specs=[
            pl.BlockSpec((_EBLK, kdim), lambda i: (i, 0)),
            pl.BlockSpec((kdim, HH), lambda i: (0, 0)),
            pl.BlockSpec((1, HH), lambda i: (0, 0)),
        ],
        out_specs=pl.BlockSpec((_EBLK, HH), lambda i: (i, 0)),
        out_shape=jax.ShapeDtypeStruct((EE, HH), jnp.float32),
    )(w, m, b)


def _cls_body(w2, wc, bc, o):
    t = jnp.sum(w2[...] * wc[...], axis=1, keepdims=True) + bc[0, 0]
    o[...] = 1.0 / (1.0 + jnp.exp(-t))


def _tc_cls(w2, wc2, bc2):
    return pl.pallas_call(
        _cls_body,
        grid=(_EGRID,),
        in_specs=[
            pl.BlockSpec((_EBLK, HH), lambda i: (i, 0)),
            pl.BlockSpec((1, HH), lambda i: (0, 0)),
            pl.BlockSpec((1, 1), lambda i: (0, 0)),
        ],
        out_specs=pl.BlockSpec((_EBLK, 1), lambda i: (i, 0)),
        out_shape=jax.ShapeDtypeStruct((EE, 1), jnp.float32),
    )(w2, wc2, bc2)


def kernel(X, edge_index, D, Wp1, bp1, Ws1, bs1, Wpe1, bpe1, Wse1, bse1,
           Wp2, bp2, Ws2, bs2, Wpe2, bpe2, Wse2, bse2, Wc, bc):
    src = edge_index[0]
    dst = edge_index[1]
    d2 = D[:, None]

    a1, b1 = Wpe1[:, :HH], Wpe1[:, HH:]
    wa1t = ((a1 + b1) * 0.5).T
    wb1t = ((b1 - a1) * 0.5).T
    a2, b2 = Wpe2[:, :HH], Wpe2[:, HH:]
    wa2t = ((a2 + b2) * 0.5).T
    wb2t = ((b2 - a2) * 0.5).T

    c0 = (bp1 + bs1)[None, :]
    c1 = (bpe1 + bse1)[None, :]
    c2 = (bp2 + bs2)[None, :]
    c3 = (bpe2 + bse2)[None, :]

    w0, aggp = _sc1(X, src, dst)
    xn, g1, g2 = _tc_node1(aggp, d2, Wp1.T, wa1t, wb1t, c0)
    u0 = _tc_edge_mm(w0, Wse1.T, c1, DF)
    w1, agg2p = _sc2(g1, g2, u0, src, dst)
    g3, g4 = _tc_node2(agg2p, d2, xn, Wp2.T, Ws2.T, c2, wa2t, wb2t)
    u1 = _tc_edge_mm(w1, Wse2.T, c3, HH)
    (w2,) = _sc3(g3, g4, u1, src, dst)
    return _tc_cls(w2, Wc, bc[None, :])[:, 0]


# speedup vs baseline: 3.0723x; 1.2467x over previous
import functools

import jax
import jax.numpy as jnp
from jax import lax
from jax.experimental import pallas as pl
from jax.experimental.pallas import tpu as pltpu
from jax.experimental.pallas import tpu_sc as plsc

NN = 10000
EE = 320000
DF = 128
HH = 32

NC = 2
NS = 16
NW = NC * NS
EPW = EE // NW
NPAD = 10240
RPT = NPAD // NS
ZR = 16

_MESH = dict(core_axis_name="c", subcore_axis_name="s")


def _sc_edge_pipeline(width, mode, with_u, with_acc, chk):
    nchunk = EPW // chk
    nquad, rem = divmod(nchunk, 4)
    assert rem <= 2

    def body(*refs):
        it = iter(refs)
        t1_hbm = next(it)
        t2_hbm = next(it)
        u_hbm = next(it) if with_u else None
        src_hbm = next(it)
        dst_hbm = next(it)
        w_hbm = next(it)
        aggp_hbm = next(it) if with_acc else None
        idx_s = ((next(it), next(it)), (next(it), next(it)))
        idx_d = ((next(it), next(it)), (next(it), next(it)))
        b1 = (next(it), next(it))
        b2 = (next(it), next(it))
        bu = (next(it), next(it)) if with_u else None
        bw = (next(it), next(it))
        sem_g = (next(it), next(it))
        sem_w = (next(it), next(it))
        sem_a = (next(it), next(it)) if with_acc else None
        zbuf = next(it) if with_acc else None
        acc = next(it) if with_acc else None

        core = lax.axis_index("c")
        sid = lax.axis_index("s")
        wid = core * NS + sid
        ebase = wid * EPW
        row0 = sid * RPT

        if with_acc:
            def zrow(i, _):
                for k in range(0, width, 16):
                    zbuf[i, pl.ds(k, 16)] = jnp.zeros((16,), jnp.float32)
                return 0
            lax.fori_loop(0, ZR, zrow, 0)
            for j in range(RPT // ZR):
                pltpu.sync_copy(zbuf, acc.at[pl.ds(row0 + j * ZR, ZR)])
            plsc.subcore_barrier()

        def fetch(s, p, c):
            fbase = ebase + c * chk
            pltpu.sync_copy(src_hbm.at[pl.ds(fbase, chk)], idx_s[s][p])
            pltpu.sync_copy(dst_hbm.at[pl.ds(fbase, chk)], idx_d[s][p])
            pltpu.async_copy(t1_hbm.at[idx_s[s][p]], b1[s], sem_g[s])
            pltpu.async_copy(t2_hbm.at[idx_d[s][p]], b2[s], sem_g[s])
            if with_u:
                pltpu.async_copy(u_hbm.at[pl.ds(fbase, chk)], bu[s], sem_g[s])

        def drain_writes(s, p):
            pltpu.make_async_copy(bw[s], w_hbm.at[pl.ds(0, chk)],
                                  sem_w[s]).wait()
            if with_acc:
                pltpu.make_async_copy(bw[s], acc.at[idx_s[s][p]],
                                      sem_a[s]).wait()

        def step(s, p, c, drain_pred, fetch_c, fetch_pred=None):
            pltpu.make_async_copy(t1_hbm.at[idx_s[s][p]], b1[s],
                                  sem_g[s]).wait()
            pltpu.make_async_copy(t2_hbm.at[idx_d[s][p]], b2[s],
                                  sem_g[s]).wait()
            if with_u:
                pltpu.make_async_copy(u_hbm.at[pl.ds(0, chk)], bu[s],
                                      sem_g[s]).wait()
            if drain_pred is True:
                drain_writes(s, p)
            elif drain_pred is not False:
                @pl.when(drain_pred)
                def _():
                    drain_writes(s, p)

            def row(i, _):
                for k in range(0, width, 16):
                    sl = (i, pl.ds(k, 16))
                    if mode == "absdiff":
                        bw[s][sl] = jnp.abs(b1[s][sl] - b2[s][sl])
                    else:
                        bw[s][sl] = jnp.maximum(
                            b1[s][sl] + b2[s][sl] + bu[s][sl], 0.0)
                return 0
            lax.fori_loop(0, chk, row, 0)

            base = ebase + c * chk
            pltpu.async_copy(bw[s], w_hbm.at[pl.ds(base, chk)], sem_w[s])
            if with_acc:
                pltpu.async_copy(bw[s], acc.at[idx_s[s][p]], sem_a[s],
                                 add=True)
            if fetch_c is not None:
                if fetch_pred is None:
                    fetch(s, 1 - p, fetch_c)
                else:
                    @pl.when(fetch_pred)
                    def _():
                        fetch(s, 1 - p, fetch_c)

        fetch(0, 0, 0)
        fetch(1, 0, 1)

        def quad(g, _):
            c0 = 4 * g
            step(0, 0, c0, g > 0, c0 + 2)
            step(1, 0, c0 + 1, g > 0, c0 + 3)
            step(0, 1, c0 + 2, True, c0 + 4, c0 + 4 < nchunk)
            step(1, 1, c0 + 3, True, c0 + 5, c0 + 5 < nchunk)
            return 0
        lax.fori_loop(0, nquad, quad, 0)
        for j in range(rem):
            step(j % 2, 0, 4 * nquad + j, True, None)
        drain_writes(0, rem % 2)
        drain_writes(1, rem % 2)

        if with_acc:
            plsc.subcore_barrier()
            pltpu.sync_copy(acc.at[pl.ds(row0, RPT)],
                            aggp_hbm.at[core, pl.ds(row0, RPT)])

    return body


def _sc_scratch(width, with_u, with_acc, chk):
    st = []
    st += [pltpu.VMEM((chk,), jnp.int32)] * 4
    st += [pltpu.VMEM((chk,), jnp.int32)] * 4
    st += [pltpu.VMEM((chk, width), jnp.float32)] * 2
    st += [pltpu.VMEM((chk, width), jnp.float32)] * 2
    if with_u:
        st += [pltpu.VMEM((chk, width), jnp.float32)] * 2
    st += [pltpu.VMEM((chk, width), jnp.float32)] * 2
    st += [pltpu.SemaphoreType.DMA] * 2
    st += [pltpu.SemaphoreType.DMA] * 2
    if with_acc:
        st += [pltpu.SemaphoreType.DMA] * 2
        st += [pltpu.VMEM((ZR, width), jnp.float32)]
        st += [pltpu.VMEM_SHARED((NPAD, width), jnp.float32)]
    return st


_sc1 = pl.kernel(
    _sc_edge_pipeline(DF, "absdiff", False, True, 40),
    out_type=[jax.ShapeDtypeStruct((EE, DF), jnp.float32),
              jax.ShapeDtypeStruct((2, NPAD, DF), jnp.float32)],
    mesh=plsc.VectorSubcoreMesh(**_MESH),
    scratch_types=_sc_scratch(DF, False, True, 40),
)

_sc2 = pl.kernel(
    _sc_edge_pipeline(HH, "relusum", True, True, 80),
    out_type=[jax.ShapeDtypeStruct((EE, HH), jnp.float32),
              jax.ShapeDtypeStruct((2, NPAD, HH), jnp.float32)],
    compiler_params=pltpu.CompilerParams(use_tc_tiling_on_sc=False),
    mesh=plsc.VectorSubcoreMesh(**_MESH),
    scratch_types=_sc_scratch(HH, True, True, 80),
)

_sc3 = pl.kernel(
    _sc_edge_pipeline(HH, "relusum", True, False, 80),
    out_type=[jax.ShapeDtypeStruct((EE, HH), jnp.float32)],
    compiler_params=pltpu.CompilerParams(use_tc_tiling_on_sc=False),
    mesh=plsc.VectorSubcoreMesh(**_MESH),
    scratch_types=_sc_scratch(HH, True, False, 80),
)


_NBLK = 1000
_NGRID = NN // _NBLK
_EBLK = 2560
_EGRID = EE // _EBLK


def _node1_body(a0, a1, d, wp, wa, wb, c0, xn_o, g1_o, g2_o):
    agg = a0[0] + a1[0]
    xn = jnp.maximum(
        jnp.dot(agg / d[...], wp[...], preferred_element_type=jnp.float32)
        + c0[...], 0.0)
    xn_o[...] = xn
    g1_o[...] = jnp.dot(xn, wa[...], preferred_element_type=jnp.float32)
    g2_o[...] = jnp.dot(xn, wb[...], preferred_element_type=jnp.float32)


def _tc_node1(aggp, d2, wp1t, wa1t, wb1t, c0):
    return pl.pallas_call(
        _node1_body,
        grid=(_NGRID,),
        in_specs=[
            pl.BlockSpec((1, _NBLK, DF), lambda i: (0, i, 0)),
            pl.BlockSpec((1, _NBLK, DF), lambda i: (1, i, 0)),
            pl.BlockSpec((_NBLK, 1), lambda i: (i, 0)),
            pl.BlockSpec((DF, HH), lambda i: (0, 0)),
            pl.BlockSpec((HH, HH), lambda i: (0, 0)),
            pl.BlockSpec((HH, HH), lambda i: (0, 0)),
            pl.BlockSpec((1, HH), lambda i: (0, 0)),
        ],
        out_specs=[
            pl.BlockSpec((_NBLK, HH), lambda i: (i, 0)),
            pl.BlockSpec((_NBLK, HH), lambda i: (i, 0)),
            pl.BlockSpec((_NBLK, HH), lambda i: (i, 0)),
        ],
        out_shape=[jax.ShapeDtypeStruct((NN, HH), jnp.float32)] * 3,
    )(aggp, aggp, d2, wp1t, wa1t, wb1t, c0)


def _node2_body(a0, a1, d, xn, wp, ws, c0, wa, wb, g3_o, g4_o):
    agg = a0[0] + a1[0]
    xn2 = jnp.maximum(
        jnp.dot(agg / d[...], wp[...], preferred_element_type=jnp.float32)
        + jnp.dot(xn[...], ws[...], preferred_element_type=jnp.float32)
        + c0[...], 0.0)
    g3_o[...] = jnp.dot(xn2, wa[...], preferred_element_type=jnp.float32)
    g4_o[...] = jnp.dot(xn2, wb[...], preferred_element_type=jnp.float32)


def _tc_node2(agg2p, d2, xn, wp2t, ws2t, c2, wa2t, wb2t):
    hblk = pl.BlockSpec((HH, HH), lambda i: (0, 0))
    return pl.pallas_call(
        _node2_body,
        grid=(_NGRID,),
        in_specs=[
            pl.BlockSpec((1, _NBLK, HH), lambda i: (0, i, 0)),
            pl.BlockSpec((1, _NBLK, HH), lambda i: (1, i, 0)),
            pl.BlockSpec((_NBLK, 1), lambda i: (i, 0)),
            pl.BlockSpec((_NBLK, HH), lambda i: (i, 0)),
            hblk, hblk,
            pl.BlockSpec((1, HH), lambda i: (0, 0)),
            hblk, hblk,
        ],
        out_specs=[
            pl.BlockSpec((_NBLK, HH), lambda i: (i, 0)),
            pl.BlockSpec((_NBLK, HH), lambda i: (i, 0)),
        ],
        out_shape=[jax.ShapeDtypeStruct((NN, HH), jnp.float32)] * 2,
    )(agg2p, agg2p, d2, xn, wp2t, ws2t, c2, wa2t, wb2t)


def _edge_mm_body(w, m, b, o):
    o[...] = jnp.dot(w[...], m[...], preferred_element_type=jnp.float32) + b[...]


def _tc_edge_mm(w, m, b, kdim):
    return pl.pallas_call(
        _edge_mm_body,
        grid=(_EGRID,),
        in_specs=[
            pl.BlockSpec((_EBLK, kdim), lambda i: (i, 0)),
            pl.BlockSpec((kdim, HH), lambda i: (0, 0)),
            pl.BlockSpec((1, HH), lambda i: (0, 0)),
        ],
        out_specs=pl.BlockSpec((_EBLK, HH), lambda i: (i, 0)),
        out_shape=jax.ShapeDtypeStruct((EE, HH), jnp.float32),
    )(w, m, b)


def _cls_body(w2, wc, bc, o):
    t = jnp.sum(w2[...] * wc[...], axis=1, keepdims=True) + bc[0, 0]
    o[...] = 1.0 / (1.0 + jnp.exp(-t))


def _tc_cls(w2, wc2, bc2):
    return pl.pallas_call(
        _cls_body,
        grid=(_EGRID,),
        in_specs=[
            pl.BlockSpec((_EBLK, HH), lambda i: (i, 0)),
            pl.BlockSpec((1, HH), lambda i: (0, 0)),
            pl.BlockSpec((1, 1), lambda i: (0, 0)),
        ],
        out_specs=pl.BlockSpec((_EBLK, 1), lambda i: (i, 0)),
        out_shape=jax.ShapeDtypeStruct((EE, 1), jnp.float32),
    )(w2, wc2, bc2)


def kernel(X, edge_index, D, Wp1, bp1, Ws1, bs1, Wpe1, bpe1, Wse1, bse1,
           Wp2, bp2, Ws2, bs2, Wpe2, bpe2, Wse2, bse2, Wc, bc):
    src = edge_index[0]
    dst = edge_index[1]
    d2 = D[:, None]

    a1, b1 = Wpe1[:, :HH], Wpe1[:, HH:]
    wa1t = ((a1 + b1) * 0.5).T
    wb1t = ((b1 - a1) * 0.5).T
    a2, b2 = Wpe2[:, :HH], Wpe2[:, HH:]
    wa2t = ((a2 + b2) * 0.5).T
    wb2t = ((b2 - a2) * 0.5).T

    c0 = (bp1 + bs1)[None, :]
    c1 = (bpe1 + bse1)[None, :]
    c2 = (bp2 + bs2)[None, :]
    c3 = (bpe2 + bse2)[None, :]

    w0, aggp = _sc1(X, X, src, dst)
    xn, g1, g2 = _tc_node1(aggp, d2, Wp1.T, wa1t, wb1t, c0)
    u0 = _tc_edge_mm(w0, Wse1.T, c1, DF)
    w1, agg2p = _sc2(g1, g2, u0, src, dst)
    g3, g4 = _tc_node2(agg2p, d2, xn, Wp2.T, Ws2.T, c2, wa2t, wb2t)
    u1 = _tc_edge_mm(w1, Wse2.T, c3, HH)
    (w2,) = _sc3(g3, g4, u1, src, dst)
    return _tc_cls(w2, Wc, bc[None, :])[:, 0]
